# Initial kernel scaffold; baseline (speedup 1.0000x reference)
import jax
import jax.numpy as jnp
from jax.experimental import pallas as pl


def kernel(x, edge_index, batch, image_features, W1, b1, W2, b2, Wi1, bi1, Wi2, bi2, Wc1, bc1, gamma, beta, Wc2, bc2):
    raise NotImplementedError("write your pallas kernel here")


# speedup vs baseline: 12.7605x; 12.7605x over previous
import functools

import jax
import jax.numpy as jnp
from jax import lax
from jax.experimental import pallas as pl
from jax.experimental.pallas import tpu as pltpu
from jax.experimental.pallas import tpu_sc as plsc

NN = 10000
NP = 10240
EE = 320000
F_IN = 128
H1 = 128
GDIM = 64
BB = 64
IMG = 1280
BN_EPS = 1e-5

NC = 2
NS = 16
NW = NC * NS
EBLK = 128
BLKS_PER_TILE = 79
EPT = BLKS_PER_TILE * EBLK
EP = EPT * NW
ROWS_PER_TILE = NP // NS
RCHUNK = ROWS_PER_TILE // 4

_MESH = plsc.VectorSubcoreMesh(
    core_axis_name="c", subcore_axis_name="s", num_cores=NC, num_subcores=NS)

_F32 = jnp.float32
_PREC = lax.Precision.HIGHEST
_SC_PARAMS = pltpu.CompilerParams(use_tc_tiling_on_sc=False)


def _zero_stage(stg_v, dcols):
    zeros16 = jnp.zeros((16,), _F32)

    @pl.loop(0, RCHUNK)
    def _(i):
        for k in range(dcols // 16):
            stg_v[i, pl.ds(k * 16, 16)] = zeros16


@functools.partial(
    pl.kernel,
    out_type=jax.ShapeDtypeStruct((NC, NP, 16), _F32),
    mesh=_MESH,
    scratch_types=[
        pltpu.VMEM((EBLK,), jnp.int32),
        pltpu.VMEM((EBLK, 16), _F32),
        pltpu.VMEM((RCHUNK, 16), _F32),
        pltpu.VMEM_SHARED((NP, 16), _F32),
    ],
    compiler_params=_SC_PARAMS,
)
def _deg_kernel(dst_hbm, out_hbm, dst_v, ones_v, stg_v, acc):
    cid = lax.axis_index("c")
    sid = lax.axis_index("s")
    wid = cid * NS + sid
    ones16 = jnp.ones((16,), _F32)

    @pl.loop(0, EBLK)
    def _(i):
        ones_v[i] = ones16

    _zero_stage(stg_v, 16)
    for c in range(4):
        pltpu.sync_copy(
            stg_v, acc.at[pl.ds(sid * ROWS_PER_TILE + c * RCHUNK, RCHUNK)])
    plsc.subcore_barrier()

    @pl.loop(0, BLKS_PER_TILE)
    def _(j):
        off = pl.multiple_of(wid * EPT + j * EBLK, EBLK)
        pltpu.sync_copy(dst_hbm.at[pl.ds(off, EBLK)], dst_v)
        pltpu.sync_copy(ones_v, acc.at[dst_v], add=True)

    plsc.subcore_barrier()
    for c in range(4):
        r0 = sid * ROWS_PER_TILE + c * RCHUNK
        pltpu.sync_copy(acc.at[pl.ds(r0, RCHUNK)], stg_v)
        pltpu.sync_copy(stg_v, out_hbm.at[cid].at[pl.ds(r0, RCHUNK)])


def _make_agg(dcols):
    @functools.partial(
        pl.kernel,
        out_type=jax.ShapeDtypeStruct((NC, NP, dcols), _F32),
        mesh=_MESH,
        scratch_types=[
            pltpu.VMEM((EBLK,), jnp.int32),
            pltpu.VMEM((EBLK,), jnp.int32),
            pltpu.VMEM((EBLK, dcols), _F32),
            pltpu.VMEM((RCHUNK, dcols), _F32),
            pltpu.VMEM_SHARED((NP, dcols), _F32),
            pltpu.SemaphoreType.DMA,
        ],
        compiler_params=_SC_PARAMS,
    )
    def agg(hs_hbm, src_hbm, dst_hbm, out_hbm, src_v, dst_v, rows_v, stg_v,
            acc, sem):
        cid = lax.axis_index("c")
        sid = lax.axis_index("s")
        wid = cid * NS + sid

        _zero_stage(stg_v, dcols)
        for c in range(4):
            pltpu.sync_copy(
                stg_v, acc.at[pl.ds(sid * ROWS_PER_TILE + c * RCHUNK, RCHUNK)])
        plsc.subcore_barrier()

        @pl.loop(0, BLKS_PER_TILE)
        def _(j):
            off = pl.multiple_of(wid * EPT + j * EBLK, EBLK)
            pltpu.sync_copy(src_hbm.at[pl.ds(off, EBLK)], src_v)
            pltpu.sync_copy(dst_hbm.at[pl.ds(off, EBLK)], dst_v)
            pltpu.async_copy(hs_hbm.at[src_v], rows_v, sem).wait()
            pltpu.sync_copy(rows_v, acc.at[dst_v], add=True)

        plsc.subcore_barrier()
        for c in range(4):
            r0 = sid * ROWS_PER_TILE + c * RCHUNK
            pltpu.sync_copy(acc.at[pl.ds(r0, RCHUNK)], stg_v)
            pltpu.sync_copy(stg_v, out_hbm.at[cid].at[pl.ds(r0, RCHUNK)])

    return agg


_agg128 = _make_agg(H1)
_agg64 = _make_agg(GDIM)


def _dinv_from(deg_ref):
    deg = deg_ref[0][:, 0:1] + deg_ref[1][:, 0:1] + 1.0
    return lax.rsqrt(deg)


def _scale1_body(deg_ref, x_ref, w1_ref, hs_ref):
    dinv = _dinv_from(deg_ref)
    h = jnp.dot(x_ref[...], w1_ref[...], precision=_PREC,
                preferred_element_type=_F32)
    hs_ref[...] = h * dinv


def _layer2_body(p_ref, hs1_ref, deg_ref, w2_ref, b1_ref, hs2_ref):
    dinv = _dinv_from(deg_ref)
    agg = p_ref[0] + p_ref[1] + hs1_ref[...]
    out1 = jnp.maximum(agg * dinv + b1_ref[...], 0.0)
    h2 = jnp.dot(out1, w2_ref[...], precision=_PREC,
                 preferred_element_type=_F32)
    hs2_ref[...] = h2 * dinv


def _head_body(q_ref, hs2_ref, deg_ref, b2_ref, batch_ref, img_ref,
               wi1_ref, bi1_ref, wi2_ref, bi2_ref, wc1_ref, bc1_ref,
               gamma_ref, beta_ref, wc2_ref, bc2_ref, out_ref):
    dinv = _dinv_from(deg_ref)
    out2 = (q_ref[0] + q_ref[1] + hs2_ref[...]) * dinv + b2_ref[...]
    iota = lax.broadcasted_iota(jnp.int32, (NP, BB), 1)
    oh = (batch_ref[...] == iota).astype(_F32)
    sums = lax.dot_general(oh, out2, (((0,), (0,)), ((), ())),
                           precision=_PREC, preferred_element_type=_F32)
    counts = jnp.sum(oh, axis=0)[:, None]
    ge = sums / jnp.maximum(counts, 1.0)
    img = jnp.maximum(
        jnp.dot(img_ref[...], wi1_ref[...], precision=_PREC,
                preferred_element_type=_F32) + bi1_ref[...], 0.0)
    ie = jnp.dot(img, wi2_ref[...], precision=_PREC,
                 preferred_element_type=_F32) + bi2_ref[...]
    comb = jnp.concatenate([ge, ie], axis=1)
    z = jnp.maximum(
        jnp.dot(comb, wc1_ref[...], precision=_PREC,
                preferred_element_type=_F32) + bc1_ref[...], 0.0)
    z = z * (gamma_ref[...] * (1.0 / (1.0 + BN_EPS) ** 0.5)) + beta_ref[...]
    out_ref[...] = jnp.dot(z, wc2_ref[...], precision=_PREC,
                           preferred_element_type=_F32) + bc2_ref[...]


_scale1 = pl.pallas_call(
    _scale1_body, out_shape=jax.ShapeDtypeStruct((NP, H1), _F32))
_layer2 = pl.pallas_call(
    _layer2_body, out_shape=jax.ShapeDtypeStruct((NP, GDIM), _F32))
_head = pl.pallas_call(
    _head_body, out_shape=jax.ShapeDtypeStruct((BB, 1), _F32))


def kernel(x, edge_index, batch, image_features, W1, b1, W2, b2,
           Wi1, bi1, Wi2, bi2, Wc1, bc1, gamma, beta, Wc2, bc2):
    x_pad = jnp.zeros((NP, F_IN), _F32).at[:NN].set(x)
    fill = jnp.full((EP - EE,), NN, jnp.int32)
    src_p = jnp.concatenate([edge_index[0], fill])
    dst_p = jnp.concatenate([edge_index[1], fill])
    batch_p = jnp.concatenate(
        [batch, jnp.full((NP - NN,), BB, jnp.int32)])[:, None]

    degp = _deg_kernel(dst_p)
    hs1 = _scale1(degp, x_pad, W1)
    p1 = _agg128(hs1, src_p, dst_p)
    hs2 = _layer2(p1, hs1, degp, W2, b1[None, :])
    p2 = _agg64(hs2, src_p, dst_p)
    out = _head(p2, hs2, degp, b2[None, :], batch_p, image_features,
                Wi1, bi1[None, :], Wi2, bi2[None, :], Wc1, bc1[None, :],
                gamma[None, :], beta[None, :], Wc2, bc2[None, :])
    return out
